# baseline (speedup 1.0000x reference)
import jax
import jax.numpy as jnp
from jax.experimental import pallas as pl
from jax.experimental.pallas import tpu as pltpu

_CRACK = 1
_KPAD = 128


def _loss_kernel(row_ref, lg_ref, col_ref, tgt_ref, vw_ref, out_ref, tile_ref):
    b = pl.program_id(0)
    n_ch = 4
    h = lg_ref.shape[0] // n_ch
    w = lg_ref.shape[2]
    for k in range(_KPAD):
        r = row_ref[b, k]
        for c in range(n_ch):
            tile_ref[c * _KPAD + k, 0] = lg_ref[c * h + r, 0]
    t = tile_ref[...]
    tc = [t[c * _KPAD:(c + 1) * _KPAD] for c in range(n_ch)]
    m = jnp.maximum(jnp.maximum(tc[0], tc[1]), jnp.maximum(tc[2], tc[3]))
    e = [jnp.exp(x - m) for x in tc]
    s = (e[0] + e[1]) + (e[2] + e[3])
    crack = e[_CRACK] * (1.0 / s)
    lane = jax.lax.broadcasted_iota(jnp.int32, crack.shape, 2)
    sel = lane == col_ref[...]
    d = crack - tgt_ref[...]
    term = jnp.where(sel, vw_ref[...] * d * d, 0.0)
    out_ref[...] = jnp.sum(term, axis=(0, 2), keepdims=True)[0]


def kernel(logits, masks, term_idx, term_tgt, term_valid, term_count):
    del masks
    b_n, c_n, h_n, w_n = logits.shape
    k_n = term_idx.shape[1]
    pad = _KPAD - k_n
    idx = jnp.pad(term_idx, ((0, 0), (0, pad)))
    tgt = jnp.pad(term_tgt, ((0, 0), (0, pad)))
    valid = jnp.pad(term_valid, ((0, 0), (0, pad)))
    rows = (idx // w_n).astype(jnp.int32)
    cols = (idx % w_n).astype(jnp.int32).reshape(b_n, _KPAD, 1, 1)
    vw = (valid / (term_count * b_n)[:, None]).reshape(b_n, _KPAD, 1, 1)
    tgt = tgt.reshape(b_n, _KPAD, 1, 1)
    lg = logits.reshape(b_n, c_n * h_n, 1, w_n)
    out = pl.pallas_call(
        _loss_kernel,
        grid=(b_n,),
        in_specs=[
            pl.BlockSpec(memory_space=pltpu.SMEM),
            pl.BlockSpec((None, c_n * h_n, 1, w_n), lambda b: (b, 0, 0, 0)),
            pl.BlockSpec((None, _KPAD, 1, 1), lambda b: (b, 0, 0, 0)),
            pl.BlockSpec((None, _KPAD, 1, 1), lambda b: (b, 0, 0, 0)),
            pl.BlockSpec((None, _KPAD, 1, 1), lambda b: (b, 0, 0, 0)),
        ],
        out_specs=pl.BlockSpec((None, 1, 1), lambda b: (b, 0, 0)),
        out_shape=jax.ShapeDtypeStruct((b_n, 1, 1), jnp.float32),
        scratch_shapes=[pltpu.VMEM((4 * _KPAD, 1, w_n), jnp.float32)],
        compiler_params=pltpu.CompilerParams(
            dimension_semantics=("parallel",),
        ),
        name="topology_loss",
    )(rows, lg, cols, tgt, vw)
    return jnp.sum(out)


# speedup vs baseline: 5.7957x; 5.7957x over previous
import jax
import jax.numpy as jnp
from jax.experimental import pallas as pl
from jax.experimental.pallas import tpu as pltpu

_CRACK = 1
_KPAD = 128


def _loss_kernel(row_ref, colc_ref, lg_ref, col_ref, tgt_ref, vw_ref,
                 out_ref, tile_ref):
    b = pl.program_id(0)
    n_ch = 4
    for k in range(_KPAD):
        r = row_ref[b, k]
        cs = pl.multiple_of(colc_ref[b, k], 128)
        r8 = pl.multiple_of((r >> 3) << 3, 8)
        rs = r & 7
        for c in range(n_ch):
            chunk = lg_ref[c, pl.ds(r8, 8), pl.ds(cs, 128)]
            tile_ref[c * _KPAD + k] = pltpu.roll(chunk, -rs, axis=0)[0:1, :]
    t = tile_ref[...]
    tc = [t[c * _KPAD:(c + 1) * _KPAD] for c in range(n_ch)]
    m = jnp.maximum(jnp.maximum(tc[0], tc[1]), jnp.maximum(tc[2], tc[3]))
    e = [jnp.exp(x - m) for x in tc]
    s = (e[0] + e[1]) + (e[2] + e[3])
    crack = e[_CRACK] * (1.0 / s)
    lane = jax.lax.broadcasted_iota(jnp.int32, crack.shape, 2)
    sel = lane == col_ref[...]
    d = crack - tgt_ref[...]
    term = jnp.where(sel, vw_ref[...] * d * d, 0.0)
    out_ref[...] = jnp.sum(term, axis=(0, 2), keepdims=True)[0]


def kernel(logits, masks, term_idx, term_tgt, term_valid, term_count):
    del masks
    b_n, c_n, h_n, w_n = logits.shape
    k_n = term_idx.shape[1]
    pad = _KPAD - k_n
    idx = jnp.pad(term_idx, ((0, 0), (0, pad)))
    tgt = jnp.pad(term_tgt, ((0, 0), (0, pad)))
    valid = jnp.pad(term_valid, ((0, 0), (0, pad)))
    rows = (idx // w_n).astype(jnp.int32)
    col = (idx % w_n).astype(jnp.int32)
    colc = col & ~jnp.int32(127)
    lanec = (col & 127).reshape(b_n, _KPAD, 1, 1)
    vw = (valid / (term_count * b_n)[:, None]).reshape(b_n, _KPAD, 1, 1)
    tgt = tgt.reshape(b_n, _KPAD, 1, 1)
    out = pl.pallas_call(
        _loss_kernel,
        grid=(b_n,),
        in_specs=[
            pl.BlockSpec(memory_space=pltpu.SMEM),
            pl.BlockSpec(memory_space=pltpu.SMEM),
            pl.BlockSpec((None, c_n, h_n, w_n), lambda b: (b, 0, 0, 0)),
            pl.BlockSpec((None, _KPAD, 1, 1), lambda b: (b, 0, 0, 0)),
            pl.BlockSpec((None, _KPAD, 1, 1), lambda b: (b, 0, 0, 0)),
            pl.BlockSpec((None, _KPAD, 1, 1), lambda b: (b, 0, 0, 0)),
        ],
        out_specs=pl.BlockSpec((None, 1, 1), lambda b: (b, 0, 0)),
        out_shape=jax.ShapeDtypeStruct((b_n, 1, 1), jnp.float32),
        scratch_shapes=[pltpu.VMEM((4 * _KPAD, 1, 128), jnp.float32)],
        compiler_params=pltpu.CompilerParams(
            dimension_semantics=("parallel",),
        ),
        name="topology_loss",
    )(rows, colc, logits, lanec, tgt, vw)
    return jnp.sum(out)
